# Initial kernel scaffold; baseline (speedup 1.0000x reference)
import jax
import jax.numpy as jnp
from jax.experimental import pallas as pl


def kernel(gene_ids, table):
    raise NotImplementedError("write your pallas kernel here")


# speedup vs baseline: 3.1682x; 3.1682x over previous
import functools

import jax
import jax.numpy as jnp
from jax import lax
from jax.experimental import pallas as pl
from jax.experimental.pallas import tpu as pltpu
from jax.experimental.pallas import tpu_sc as plsc

EMBED_DIM = 64
NUM_CORES = 2
NUM_SUBCORES = 16
NUM_WORKERS = NUM_CORES * NUM_SUBCORES
CHUNK = 128


@functools.partial(jax.jit, static_argnames=("total",))
def _gather_flat(idx_flat, table, total):
    b_per_w = total // NUM_WORKERS
    n_chunks = b_per_w // CHUNK
    mesh = plsc.VectorSubcoreMesh(
        core_axis_name="c",
        subcore_axis_name="s",
        num_cores=NUM_CORES,
        num_subcores=NUM_SUBCORES,
    )

    @functools.partial(
        pl.kernel,
        mesh=mesh,
        out_type=jax.ShapeDtypeStruct((total, EMBED_DIM), jnp.float32),
        scratch_types=[
            pltpu.VMEM((CHUNK,), jnp.int32),
            pltpu.VMEM((CHUNK, EMBED_DIM), jnp.float32),
            pltpu.SemaphoreType.DMA,
        ],
        compiler_params=pltpu.CompilerParams(use_tc_tiling_on_sc=False),
    )
    def k(table_hbm, idx_hbm, out_hbm, idx_v, rows_v, sem):
        wid = lax.axis_index("s") * NUM_CORES + lax.axis_index("c")
        base = wid * b_per_w

        def body(i, carry):
            off = base + i * CHUNK
            pltpu.sync_copy(idx_hbm.at[pl.ds(off, CHUNK)], idx_v)
            pltpu.async_copy(table_hbm.at[idx_v], rows_v, sem).wait()
            pltpu.sync_copy(rows_v, out_hbm.at[pl.ds(off, CHUNK)])
            return carry

        lax.fori_loop(0, n_chunks, body, 0)

    return k(table, idx_flat)


def kernel(gene_ids, table):
    batch, seq = gene_ids.shape
    total = batch * seq
    idx_flat = gene_ids.reshape(total).astype(jnp.int32)
    emb = _gather_flat(idx_flat, table, total)
    return (gene_ids, emb.reshape(batch, seq, EMBED_DIM))


# speedup vs baseline: 4.1982x; 1.3251x over previous
import functools

import jax
import jax.numpy as jnp
from jax import lax
from jax.experimental import pallas as pl
from jax.experimental.pallas import tpu as pltpu
from jax.experimental.pallas import tpu_sc as plsc

EMBED_DIM = 64
NUM_CORES = 2
NUM_SUBCORES = 16
NUM_WORKERS = NUM_CORES * NUM_SUBCORES
CHUNK = 128
NBUF = 4
BLOCK = CHUNK * NBUF


@functools.partial(jax.jit, static_argnames=("total",))
def _gather_flat(idx_flat, table, total):
    b_per_w = total // NUM_WORKERS
    n_blocks = b_per_w // BLOCK
    mesh = plsc.VectorSubcoreMesh(
        core_axis_name="c",
        subcore_axis_name="s",
        num_cores=NUM_CORES,
        num_subcores=NUM_SUBCORES,
    )

    @functools.partial(
        pl.kernel,
        mesh=mesh,
        out_type=jax.ShapeDtypeStruct((total, EMBED_DIM), jnp.float32),
        scratch_types=[
            pltpu.VMEM((b_per_w,), jnp.int32),
            pltpu.VMEM((2, BLOCK, EMBED_DIM), jnp.float32),
            pltpu.SemaphoreType.DMA,
            pltpu.SemaphoreType.DMA,
            pltpu.SemaphoreType.DMA,
            pltpu.SemaphoreType.DMA,
        ],
        compiler_params=pltpu.CompilerParams(use_tc_tiling_on_sc=False),
    )
    def k(table_hbm, idx_hbm, out_hbm, idx_v, rows_v, gsem0, gsem1, ssem0, ssem1):
        wid = lax.axis_index("s") * NUM_CORES + lax.axis_index("c")
        base = wid * b_per_w
        gsem = (gsem0, gsem1)
        ssem = (ssem0, ssem1)

        pltpu.sync_copy(idx_hbm.at[pl.ds(base, b_per_w)], idx_v)

        def fire_gathers(j, p):
            handles = []
            for b in range(NBUF):
                idx_slice = idx_v.at[pl.ds(j * BLOCK + b * CHUNK, CHUNK)]
                handles.append(
                    pltpu.async_copy(
                        table_hbm.at[idx_slice],
                        rows_v.at[p, pl.ds(b * CHUNK, CHUNK)],
                        gsem[p],
                    )
                )
            return handles

        def fire_store(j, p):
            pltpu.async_copy(
                rows_v.at[p], out_hbm.at[pl.ds(base + j * BLOCK, BLOCK)], ssem[p]
            )

        def drain_store(p):
            pltpu.make_async_copy(
                rows_v.at[p], out_hbm.at[pl.ds(base, BLOCK)], ssem[p]
            ).wait()

        def do_pair(j, first):
            if not first:
                drain_store(0)
            hg0 = fire_gathers(j, 0)
            if not first:
                drain_store(1)
            hg1 = fire_gathers(j + 1, 1)
            for h in hg0:
                h.wait()
            fire_store(j, 0)
            for h in hg1:
                h.wait()
            fire_store(j + 1, 1)

        do_pair(0, first=True)

        def loop_body(t, carry):
            do_pair(2 + 2 * t, first=False)
            return carry

        lax.fori_loop(0, (n_blocks - 2) // 2, loop_body, 0)
        drain_store(0)
        drain_store(1)

    return k(table, idx_flat)


def kernel(gene_ids, table):
    batch, seq = gene_ids.shape
    total = batch * seq
    idx_flat = gene_ids.reshape(total).astype(jnp.int32)
    emb = _gather_flat(idx_flat, table, total)
    return (gene_ids, emb.reshape(batch, seq, EMBED_DIM))


# speedup vs baseline: 4.2075x; 1.0022x over previous
import functools

import jax
import jax.numpy as jnp
from jax import lax
from jax.experimental import pallas as pl
from jax.experimental.pallas import tpu as pltpu
from jax.experimental.pallas import tpu_sc as plsc

EMBED_DIM = 64
SEQ = 200
NUM_CORES = 2
NUM_SUBCORES = 16
NUM_WORKERS = NUM_CORES * NUM_SUBCORES
BATCHES_PER_BLOCK = 2
BLOCK = BATCHES_PER_BLOCK * SEQ
ROW_CHUNKS = ((0, 128), (128, 72))


@functools.partial(jax.jit, static_argnames=("batch",))
def _gather_3d(idx_flat, table, batch):
    total = batch * SEQ
    b_per_w = total // NUM_WORKERS
    batches_per_w = batch // NUM_WORKERS
    n_blocks = batches_per_w // BATCHES_PER_BLOCK
    mesh = plsc.VectorSubcoreMesh(
        core_axis_name="c",
        subcore_axis_name="s",
        num_cores=NUM_CORES,
        num_subcores=NUM_SUBCORES,
    )

    @functools.partial(
        pl.kernel,
        mesh=mesh,
        out_type=jax.ShapeDtypeStruct((batch, SEQ, EMBED_DIM), jnp.float32),
        scratch_types=[
            pltpu.VMEM((b_per_w,), jnp.int32),
            pltpu.VMEM((2, BATCHES_PER_BLOCK, SEQ, EMBED_DIM), jnp.float32),
            pltpu.SemaphoreType.DMA,
            pltpu.SemaphoreType.DMA,
            pltpu.SemaphoreType.DMA,
            pltpu.SemaphoreType.DMA,
        ],
        compiler_params=pltpu.CompilerParams(use_tc_tiling_on_sc=False),
    )
    def k(table_hbm, idx_hbm, out_hbm, idx_v, rows_v, gsem0, gsem1, ssem0, ssem1):
        wid = lax.axis_index("s") * NUM_CORES + lax.axis_index("c")
        base = wid * b_per_w
        batch_base = wid * batches_per_w
        gsem = (gsem0, gsem1)
        ssem = (ssem0, ssem1)

        pltpu.sync_copy(idx_hbm.at[pl.ds(base, b_per_w)], idx_v)

        def fire_gathers(j, p):
            handles = []
            for b2 in range(BATCHES_PER_BLOCK):
                for off, sz in ROW_CHUNKS:
                    tok = j * BLOCK + b2 * SEQ + off
                    idx_slice = idx_v.at[pl.ds(tok, sz)]
                    handles.append(
                        pltpu.async_copy(
                            table_hbm.at[idx_slice],
                            rows_v.at[p, b2, pl.ds(off, sz)],
                            gsem[p],
                        )
                    )
            return handles

        def fire_store(j, p):
            pltpu.async_copy(
                rows_v.at[p],
                out_hbm.at[pl.ds(batch_base + j * BATCHES_PER_BLOCK,
                                 BATCHES_PER_BLOCK)],
                ssem[p],
            )

        def drain_store(p):
            pltpu.make_async_copy(
                rows_v.at[p],
                out_hbm.at[pl.ds(batch_base, BATCHES_PER_BLOCK)],
                ssem[p],
            ).wait()

        def do_pair(j, first):
            if not first:
                drain_store(0)
            hg0 = fire_gathers(j, 0)
            if not first:
                drain_store(1)
            hg1 = fire_gathers(j + 1, 1)
            for h in hg0:
                h.wait()
            fire_store(j, 0)
            for h in hg1:
                h.wait()
            fire_store(j + 1, 1)

        do_pair(0, first=True)

        def loop_body(t, carry):
            do_pair(2 + 2 * t, first=False)
            return carry

        lax.fori_loop(0, (n_blocks - 2) // 2, loop_body, 0)
        drain_store(0)
        drain_store(1)

    return k(table, idx_flat)


def kernel(gene_ids, table):
    batch, seq = gene_ids.shape
    idx_flat = gene_ids.reshape(batch * seq).astype(jnp.int32)
    emb = _gather_3d(idx_flat, table, batch)
    return (gene_ids, emb)


# speedup vs baseline: 4.6304x; 1.1005x over previous
import functools

import jax
import jax.numpy as jnp
from jax import lax
from jax.experimental import pallas as pl
from jax.experimental.pallas import tpu as pltpu
from jax.experimental.pallas import tpu_sc as plsc

EMBED_DIM = 64
PAD_DIM = 128
SEQ = 200
NUM_CORES = 2
NUM_SUBCORES = 16
NUM_WORKERS = NUM_CORES * NUM_SUBCORES
ROW_CHUNKS = ((0, 128), (128, 72))


@functools.partial(jax.jit, static_argnames=("batch",))
def _gather_native(idx_flat, table_p, batch):
    total = batch * SEQ
    b_per_w = total // NUM_WORKERS
    batches_per_w = batch // NUM_WORKERS
    mesh = plsc.VectorSubcoreMesh(
        core_axis_name="c",
        subcore_axis_name="s",
        num_cores=NUM_CORES,
        num_subcores=NUM_SUBCORES,
    )

    @functools.partial(
        pl.kernel,
        mesh=mesh,
        out_type=jax.ShapeDtypeStruct((batch, SEQ, EMBED_DIM), jnp.float32),
        scratch_types=[
            pltpu.VMEM((total // NUM_WORKERS,), jnp.int32),
            pltpu.VMEM((2, SEQ, PAD_DIM), jnp.float32),
            pltpu.VMEM((2, 1, SEQ, EMBED_DIM), jnp.float32),
            pltpu.SemaphoreType.DMA,
            pltpu.SemaphoreType.DMA,
            pltpu.SemaphoreType.DMA,
            pltpu.SemaphoreType.DMA,
        ],
    )
    def k(table_hbm, idx_hbm, out_hbm, idx_v, rows_a, rows_b,
          gsem0, gsem1, ssem0, ssem1):
        wid = lax.axis_index("s") * NUM_CORES + lax.axis_index("c")
        base = wid * b_per_w
        batch_base = wid * batches_per_w
        gsem = (gsem0, gsem1)
        ssem = (ssem0, ssem1)

        pltpu.sync_copy(idx_hbm.at[pl.ds(base, b_per_w)], idx_v)

        def fire_gathers(j, p):
            handles = []
            for off, sz in ROW_CHUNKS:
                handles.append(
                    pltpu.async_copy(
                        table_hbm.at[idx_v.at[pl.ds(j * SEQ + off, sz)]],
                        rows_a.at[p, pl.ds(off, sz)],
                        gsem[p],
                    )
                )
            return handles

        def repack(p):
            def body(r, carry):
                for c in range(EMBED_DIM // 16):
                    rows_b[p, 0, r, pl.ds(c * 16, 16)] = (
                        rows_a[p, r, pl.ds(c * 16, 16)]
                    )
                return carry
            lax.fori_loop(0, SEQ, body, 0)

        def fire_store(j, p):
            pltpu.async_copy(
                rows_b.at[p], out_hbm.at[pl.ds(batch_base + j, 1)], ssem[p]
            )

        def drain_store(p):
            pltpu.make_async_copy(
                rows_b.at[p], out_hbm.at[pl.ds(batch_base, 1)], ssem[p]
            ).wait()

        def do_pair(j, first):
            hg0 = fire_gathers(j, 0)
            hg1 = fire_gathers(j + 1, 1)
            for h in hg0:
                h.wait()
            if not first:
                drain_store(0)
            repack(0)
            fire_store(j, 0)
            for h in hg1:
                h.wait()
            if not first:
                drain_store(1)
            repack(1)
            fire_store(j + 1, 1)

        do_pair(0, first=True)

        def loop_body(t, carry):
            do_pair(2 + 2 * t, first=False)
            return carry

        lax.fori_loop(0, (batches_per_w - 2) // 2, loop_body, 0)
        drain_store(0)
        drain_store(1)

    return k(table_p, idx_flat)


def kernel(gene_ids, table):
    batch, seq = gene_ids.shape
    idx_flat = gene_ids.reshape(batch * seq).astype(jnp.int32)
    table_p = jnp.pad(table, ((0, 0), (0, PAD_DIM - EMBED_DIM)))
    emb = _gather_native(idx_flat, table_p, batch)
    return (gene_ids, emb)
